# Initial kernel scaffold; baseline (speedup 1.0000x reference)
import jax
import jax.numpy as jnp
from jax.experimental import pallas as pl


def kernel(vectors, node_feats, node_specie, radial_embedding, senders, receivers, W_up_s, W_up_v, W_mlp0, W_mlp1, W_mlp2, W_mlp3, Ws_skip, Wv_skip, Wd_s, Wd_v):
    raise NotImplementedError("write your pallas kernel here")


# speedup vs baseline: 9.6394x; 9.6394x over previous
import functools
import math

import jax
import jax.numpy as jnp
from jax import lax
from jax.experimental import pallas as pl
from jax.experimental.pallas import tpu as pltpu
from jax.experimental.pallas import tpu_sc as plsc

N = 50000
E = 800000
E_PAD = 819200
T_E = E_PAD // 128
K_GROUP = 8
G_TOTAL = T_E // K_GROUP
GROUP_E = K_GROUP * 128
NPAD = 50048
ROWS_PER_TILE = NPAD // 16

_SQ3 = math.sqrt(3.0)
_SQ75 = math.sqrt(7.5)

BN = 2000
BE = 2048


def _silu(x):
    return x / (1.0 + jnp.exp(-x))


def _up_body(nf_ref, wus_ref, wuv_ref, out_ref):
    nf = nf_ref[...]
    xs = nf[:, :32]
    out_ref[:, :32] = jnp.dot(xs, wus_ref[...],
                              preferred_element_type=jnp.float32) * (1.0 / math.sqrt(32.0))
    wv = wuv_ref[...]
    for c in range(3):
        xv = nf[:, 32 + 16 * c:48 + 16 * c]
        out_ref[:, 32 + 16 * c:48 + 16 * c] = jnp.dot(
            xv, wv, preferred_element_type=jnp.float32) * (1.0 / math.sqrt(16.0))


def _linear_up(nf_prep, w_up_s, w_up_v):
    bn = BN
    return pl.pallas_call(
        _up_body,
        grid=(N // bn,),
        in_specs=[
            pl.BlockSpec((bn, 80), lambda i: (i, 0)),
            pl.BlockSpec((32, 32), lambda i: (0, 0)),
            pl.BlockSpec((16, 16), lambda i: (0, 0)),
        ],
        out_specs=pl.BlockSpec((bn, 80), lambda i: (i, 0)),
        out_shape=jax.ShapeDtypeStruct((N, 80), jnp.float32),
    )(nf_prep, w_up_s, w_up_v)


def _gather_body(u_hbm, idx2d_hbm, out_hbm, idx_v, rows_v, sem):
    cid = lax.axis_index("c")
    sid = lax.axis_index("s")
    wid = sid * 2 + cid
    n_g = G_TOTAL // 32
    base_g = wid * n_g

    def grp(g, carry):
        t0 = (base_g + g) * K_GROUP
        pltpu.sync_copy(idx2d_hbm.at[pl.ds(t0, K_GROUP)], idx_v)
        cps = [
            pltpu.async_copy(u_hbm.at[idx_v.at[j]],
                             rows_v.at[pl.ds(j * 128, 128)], sem)
            for j in range(K_GROUP)
        ]
        for cp in cps:
            cp.wait()
        pltpu.sync_copy(rows_v, out_hbm.at[pl.ds(t0 * 128, GROUP_E)])
        return carry

    lax.fori_loop(0, n_g, grp, 0)


def _sc_gather(u, senders2d):
    mesh = plsc.VectorSubcoreMesh(core_axis_name="c", subcore_axis_name="s")
    fn = pl.kernel(
        _gather_body,
        out_type=jax.ShapeDtypeStruct((E_PAD, 80), jnp.float32),
        mesh=mesh,
        compiler_params=pltpu.CompilerParams(use_tc_tiling_on_sc=False),
        scratch_types=[
            pltpu.VMEM((K_GROUP, 128), jnp.int32),
            pltpu.VMEM((GROUP_E, 80), jnp.float32),
            pltpu.SemaphoreType.DMA,
        ],
    )
    return fn(u, senders2d)


def _edge_body(g_ref, vec_ref, rad_ref, w0_ref, w1_ref, w2_ref, w3_ref,
               wds_ref, wdv_ref, o0, o1, o2, o3, o4, o5):
    g = g_ref[...]
    ms = g[:, :32]
    mv = [g[:, 32:48], g[:, 48:64], g[:, 64:80]]
    v = vec_ref[...]
    vx, vy, vz = v[:, 0:1], v[:, 1:2], v[:, 2:3]
    r2 = vx * vx + vy * vy + vz * vz
    r = jnp.sqrt(r2)
    nz = r > 0.0
    inv = jnp.where(nz, 1.0, 0.0) / jnp.where(nz, r, 1.0)
    u = [vx * inv, vy * inv, vz * inv]

    h = rad_ref[...]
    h = _silu(jnp.dot(h, w0_ref[...], preferred_element_type=jnp.float32)
              * (1.0 / math.sqrt(8.0)))
    h = _silu(jnp.dot(h, w1_ref[...], preferred_element_type=jnp.float32) * 0.125)
    h = _silu(jnp.dot(h, w2_ref[...], preferred_element_type=jnp.float32) * 0.125)
    mix = jnp.dot(h, w3_ref[...], preferred_element_type=jnp.float32) * 0.125
    mix = jnp.where(nz, mix, 0.0)

    mix_s1 = mix[:, :32]
    mix_s2 = mix[:, 32:48]
    mix_lo = mix[:, 48:64]
    mix_mid = mix[:, 64:96]
    mix_hi = mix[:, 96:112]

    wds = wds_ref[...]
    wdv = wdv_ref[...]

    s_dot = u[0] * mv[0] + u[1] * mv[1] + u[2] * mv[2]
    tp_vs = _SQ3 * s_dot

    outs = (jnp.dot(ms * mix_s1, wds[:32, :], preferred_element_type=jnp.float32)
            + jnp.dot(tp_vs * mix_s2, wds[32:48, :],
                      preferred_element_type=jnp.float32)) * (1.0 / math.sqrt(48.0))
    o0[...] = outs[:, :16]
    o1[...] = outs[:, 16:32]
    o2[...] = outs[:, 32:48]

    p_shared = jnp.dot(ms * mix_mid, wdv[16:48, :],
                       preferred_element_type=jnp.float32)
    q_shared = jnp.dot(s_dot * mix_hi, wdv[48:64, :],
                       preferred_element_type=jnp.float32)
    outv_refs = [o3, o4, o5]
    for c in range(3):
        t_c = jnp.dot(mv[c] * mix_lo, wdv[:16, :],
                      preferred_element_type=jnp.float32)
        r_c = jnp.dot(mv[c] * mix_hi, wdv[48:64, :],
                      preferred_element_type=jnp.float32)
        out_v_c = (t_c + (_SQ3 * u[c]) * p_shared
                   + _SQ75 * (u[c] * q_shared - r_c * (1.0 / 3.0)))
        outv_refs[c][...] = out_v_c * (1.0 / math.sqrt(64.0))


def _edge_compute(g, vectors, radial, w0, w1, w2, w3, wds, wdv):
    be = BE
    o16 = pl.BlockSpec((be, 16), lambda i: (i, 0))
    return pl.pallas_call(
        _edge_body,
        grid=(E_PAD // be,),
        in_specs=[
            pl.BlockSpec((be, 80), lambda i: (i, 0)),
            pl.BlockSpec((be, 3), lambda i: (i, 0)),
            pl.BlockSpec((be, 8), lambda i: (i, 0)),
            pl.BlockSpec((8, 64), lambda i: (0, 0)),
            pl.BlockSpec((64, 64), lambda i: (0, 0)),
            pl.BlockSpec((64, 64), lambda i: (0, 0)),
            pl.BlockSpec((64, 112), lambda i: (0, 0)),
            pl.BlockSpec((48, 48), lambda i: (0, 0)),
            pl.BlockSpec((64, 16), lambda i: (0, 0)),
        ],
        out_specs=[o16] * 6,
        out_shape=[jax.ShapeDtypeStruct((E_PAD, 16), jnp.float32)] * 6,
    )(g, vectors, radial, w0, w1, w2, w3, wds, wdv)


def _scatter_pass(msg_hbm, recv2d_hbm, zeros_hbm, agg_hbm, slab, idx_v, msg_v):
    tid = lax.axis_index("s")
    row0 = tid * ROWS_PER_TILE
    pltpu.sync_copy(zeros_hbm, slab.at[pl.ds(row0, ROWS_PER_TILE)])
    plsc.subcore_barrier()

    n_g = G_TOTAL // 16
    base_g = tid * n_g

    def grp(g, carry):
        t0 = (base_g + g) * K_GROUP
        pltpu.sync_copy(recv2d_hbm.at[pl.ds(t0, K_GROUP)], idx_v)
        pltpu.sync_copy(msg_hbm.at[pl.ds(t0 * 128, GROUP_E)], msg_v)
        for j in range(K_GROUP):
            pltpu.sync_copy(msg_v.at[pl.ds(j * 128, 128)],
                            slab.at[idx_v.at[j]], add=True)
        return carry

    lax.fori_loop(0, n_g, grp, 0)
    plsc.subcore_barrier()
    pltpu.sync_copy(slab.at[pl.ds(row0, ROWS_PER_TILE)],
                    agg_hbm.at[pl.ds(row0, ROWS_PER_TILE)])


def _scatter_body(s0, s1, s2, v0, v1, v2, recv2d, zeros,
                  a0, a1, a2, a3, a4, a5, slab, idx_v, msg_v):
    cid = lax.axis_index("c")
    pairs0 = [(s0, a0), (s1, a1), (s2, a2)]
    pairs1 = [(v0, a3), (v1, a4), (v2, a5)]
    for p in range(3):
        m0, g0 = pairs0[p]
        m1, g1 = pairs1[p]

        @pl.when(cid == 0)
        def _():
            _scatter_pass(m0, recv2d, zeros, g0, slab, idx_v, msg_v)

        @pl.when(cid == 1)
        def _():
            _scatter_pass(m1, recv2d, zeros, g1, slab, idx_v, msg_v)


def _sc_scatter(msgs, recv2d, zeros_tile):
    mesh = plsc.VectorSubcoreMesh(core_axis_name="c", subcore_axis_name="s")
    fn = pl.kernel(
        _scatter_body,
        out_type=[jax.ShapeDtypeStruct((NPAD, 16), jnp.float32)] * 6,
        mesh=mesh,
        compiler_params=pltpu.CompilerParams(use_tc_tiling_on_sc=False),
        scratch_types=[
            pltpu.VMEM_SHARED((NPAD, 16), jnp.float32),
            pltpu.VMEM((K_GROUP, 128), jnp.int32),
            pltpu.VMEM((GROUP_E, 16), jnp.float32),
        ],
    )
    return fn(*msgs, recv2d, zeros_tile)


def _node_body(a0, a1, a2, a3, a4, a5, nf_ref, sp_ref, wsc_ref, wvc_ref,
               os0, os1, ov0, ov1, ov2):
    nf = nf_ref[...]
    xs = nf[:, :32]
    sp = sp_ref[...]
    bsz = sp.shape[0]
    iota = lax.broadcasted_iota(jnp.int32, (bsz, 10), 1)
    ph = (sp == iota).astype(jnp.float32)

    ys = jnp.dot(xs, wsc_ref[...], preferred_element_type=jnp.float32) \
        * (1.0 / math.sqrt(32.0))
    sks = ph[:, 0:1] * ys[:, 0:48]
    for s in range(1, 10):
        sks = sks + ph[:, s:s + 1] * ys[:, 48 * s:48 * s + 48]

    wvc = wvc_ref[...]
    skv = []
    for c in range(3):
        yv = jnp.dot(nf[:, 32 + 16 * c:48 + 16 * c], wvc,
                     preferred_element_type=jnp.float32) * (1.0 / math.sqrt(16.0))
        acc = ph[:, 0:1] * yv[:, 0:16]
        for s in range(1, 10):
            acc = acc + ph[:, s:s + 1] * yv[:, 16 * s:16 * s + 16]
        skv.append(acc)

    inv_sq = 1.0 / math.sqrt(16.0)
    hs1 = a0[...] * inv_sq + sks[:, :16]
    hs2 = a1[...] * inv_sq + sks[:, 16:32]
    hs3 = a2[...] * inv_sq + sks[:, 32:48]
    os0[...] = jnp.maximum(_silu(hs1), 0.0)
    os1[...] = jnp.maximum(_silu(hs2), 0.0)
    gates = _silu(hs3)
    aggv = [a3, a4, a5]
    outv = [ov0, ov1, ov2]
    for c in range(3):
        hv = aggv[c][...] * inv_sq + skv[c]
        outv[c][...] = jnp.maximum(hv * gates, 0.0)


def _node_final(aggs, nf_prep, specie2d, ws_cat, wv_cat):
    bn = BN
    a16 = pl.BlockSpec((bn, 16), lambda i: (i, 0))
    return pl.pallas_call(
        _node_body,
        grid=(N // bn,),
        in_specs=[a16] * 6 + [
            pl.BlockSpec((bn, 80), lambda i: (i, 0)),
            pl.BlockSpec((bn, 1), lambda i: (i, 0)),
            pl.BlockSpec((32, 480), lambda i: (0, 0)),
            pl.BlockSpec((16, 160), lambda i: (0, 0)),
        ],
        out_specs=[a16] * 5,
        out_shape=[jax.ShapeDtypeStruct((N, 16), jnp.float32)] * 5,
    )(*aggs, nf_prep, specie2d, ws_cat, wv_cat)


def kernel(vectors, node_feats, node_specie, radial_embedding, senders,
           receivers, W_up_s, W_up_v, W_mlp0, W_mlp1, W_mlp2, W_mlp3,
           Ws_skip, Wv_skip, Wd_s, Wd_v):
    nf_prep = jnp.concatenate(
        [node_feats[:, :32],
         node_feats[:, 32:].reshape(N, 16, 3).transpose(0, 2, 1).reshape(N, 48)],
        axis=1)
    pad_e = E_PAD - E
    senders2d = jnp.concatenate(
        [senders.astype(jnp.int32),
         jnp.zeros((pad_e,), jnp.int32)]).reshape(T_E, 128)
    recv2d = jnp.concatenate(
        [receivers.astype(jnp.int32),
         jnp.full((pad_e,), N, jnp.int32)]).reshape(T_E, 128)
    vectors = jnp.pad(vectors, ((0, pad_e), (0, 0)))
    radial_embedding = jnp.pad(radial_embedding, ((0, pad_e), (0, 0)))
    specie2d = node_specie.astype(jnp.int32).reshape(N, 1)
    ws_cat = Ws_skip.transpose(1, 0, 2).reshape(32, 480)
    wv_cat = Wv_skip.transpose(1, 0, 2).reshape(16, 160)
    zeros_tile = jnp.zeros((ROWS_PER_TILE, 16), jnp.float32)

    u = _linear_up(nf_prep, W_up_s, W_up_v)
    g = _sc_gather(u, senders2d)
    msgs = _edge_compute(g, vectors, radial_embedding,
                         W_mlp0, W_mlp1, W_mlp2, W_mlp3, Wd_s, Wd_v)
    aggs = _sc_scatter(msgs, recv2d, zeros_tile)
    aggs = [a[:N] for a in aggs]
    outs = _node_final(aggs, nf_prep, specie2d, ws_cat, wv_cat)
    os0, os1, ov0, ov1, ov2 = outs
    out_v = jnp.stack([ov0, ov1, ov2], axis=-1).reshape(N, 48)
    return jnp.concatenate([os0, os1, out_v], axis=1)


# speedup vs baseline: 15.6724x; 1.6259x over previous
import functools
import math

import jax
import jax.numpy as jnp
from jax import lax
from jax.experimental import pallas as pl
from jax.experimental.pallas import tpu as pltpu
from jax.experimental.pallas import tpu_sc as plsc

N = 50000
E = 800000
E_PAD = 819200
T_E = E_PAD // 128
KG_G = 4
KG_S = 8
NPAD = 50048
ROWS_PER_TILE = NPAD // 16

_SQ3 = math.sqrt(3.0)
_SQ75 = math.sqrt(7.5)

BN = 2000
BE = 2048


def _silu(x):
    return x / (1.0 + jnp.exp(-x))


def _up_body(nf_ref, wus_ref, wuv_ref, out_ref):
    nf = nf_ref[...]
    xs = nf[:, :32]
    out_ref[:, :32] = jnp.dot(xs, wus_ref[...],
                              preferred_element_type=jnp.float32) * (1.0 / math.sqrt(32.0))
    wv = wuv_ref[...]
    for c in range(3):
        xv = nf[:, 32 + 16 * c:48 + 16 * c]
        out_ref[:, 32 + 16 * c:48 + 16 * c] = jnp.dot(
            xv, wv, preferred_element_type=jnp.float32) * (1.0 / math.sqrt(16.0))
    out_ref[:, 80:128] = jnp.zeros((nf.shape[0], 48), jnp.float32)


def _linear_up(nf_prep, w_up_s, w_up_v):
    bn = BN
    return pl.pallas_call(
        _up_body,
        grid=(N // bn,),
        in_specs=[
            pl.BlockSpec((bn, 81), lambda i: (i, 0)),
            pl.BlockSpec((32, 32), lambda i: (0, 0)),
            pl.BlockSpec((16, 16), lambda i: (0, 0)),
        ],
        out_specs=pl.BlockSpec((bn, 128), lambda i: (i, 0)),
        out_shape=jax.ShapeDtypeStruct((N, 128), jnp.float32),
    )(nf_prep, w_up_s, w_up_v)


def _gather_body(u_hbm, idx2d_hbm, vr_hbm, out_hbm, idx_v, rows_v, vr_v, sem):
    cid = lax.axis_index("c")
    sid = lax.axis_index("s")
    wid = sid * 2 + cid
    n_g = T_E // KG_G // 32
    base_g = wid * n_g

    def grp(g, carry):
        t0 = (base_g + g) * KG_G
        off = t0 * 128
        pltpu.sync_copy(idx2d_hbm.at[pl.ds(t0, KG_G)], idx_v)
        cps = [
            pltpu.async_copy(u_hbm.at[idx_v.at[j]],
                             rows_v.at[pl.ds(j * 128, 128)], sem)
            for j in range(KG_G)
        ]
        pltpu.sync_copy(vr_hbm.at[pl.ds(off // 8, KG_G * 16)], vr_v)
        for cp in cps:
            cp.wait()

        def inj(r, carry2):
            for c in range(8):
                rows_v[r * 8 + c, pl.ds(80, 16)] = vr_v[r, pl.ds(16 * c, 16)]
            return carry2

        lax.fori_loop(0, KG_G * 16, inj, 0)
        pltpu.sync_copy(rows_v, out_hbm.at[pl.ds(off, KG_G * 128)])
        return carry

    lax.fori_loop(0, n_g, grp, 0)


def _sc_gather(u, senders2d, vr):
    mesh = plsc.VectorSubcoreMesh(core_axis_name="c", subcore_axis_name="s")
    fn = pl.kernel(
        _gather_body,
        out_type=jax.ShapeDtypeStruct((E_PAD, 128), jnp.float32),
        mesh=mesh,
        compiler_params=pltpu.CompilerParams(use_tc_tiling_on_sc=False),
        scratch_types=[
            pltpu.VMEM((KG_G, 128), jnp.int32),
            pltpu.VMEM((KG_G * 128, 128), jnp.float32),
            pltpu.VMEM((KG_G * 16, 128), jnp.float32),
            pltpu.SemaphoreType.DMA,
        ],
    )
    return fn(u, senders2d, vr)


def _edge_body(g_ref, w0_ref, w1_ref, w2_ref, w3_ref, wds_ref, wdv_ref, out_ref):
    g = g_ref[...]
    ms = g[:, :32]
    mv = [g[:, 32:48], g[:, 48:64], g[:, 64:80]]
    vx, vy, vz = g[:, 80:81], g[:, 81:82], g[:, 82:83]
    r2 = vx * vx + vy * vy + vz * vz
    r = jnp.sqrt(r2)
    nz = r > 0.0
    inv = jnp.where(nz, 1.0, 0.0) / jnp.where(nz, r, 1.0)
    u = [vx * inv, vy * inv, vz * inv]

    h = g[:, 83:91]
    h = _silu(jnp.dot(h, w0_ref[...], preferred_element_type=jnp.float32)
              * (1.0 / math.sqrt(8.0)))
    h = _silu(jnp.dot(h, w1_ref[...], preferred_element_type=jnp.float32) * 0.125)
    h = _silu(jnp.dot(h, w2_ref[...], preferred_element_type=jnp.float32) * 0.125)
    mix = jnp.dot(h, w3_ref[...], preferred_element_type=jnp.float32) * 0.125
    mix = jnp.where(nz, mix, 0.0)

    mix_s1 = mix[:, :32]
    mix_s2 = mix[:, 32:48]
    mix_lo = mix[:, 48:64]
    mix_mid = mix[:, 64:96]
    mix_hi = mix[:, 96:112]

    wds = wds_ref[...]
    wdv = wdv_ref[...]

    s_dot = u[0] * mv[0] + u[1] * mv[1] + u[2] * mv[2]
    tp_vs = _SQ3 * s_dot

    outs = (jnp.dot(ms * mix_s1, wds[:32, :], preferred_element_type=jnp.float32)
            + jnp.dot(tp_vs * mix_s2, wds[32:48, :],
                      preferred_element_type=jnp.float32)) * (1.0 / math.sqrt(48.0))
    out_ref[:, 0:48] = outs

    p_shared = jnp.dot(ms * mix_mid, wdv[16:48, :],
                       preferred_element_type=jnp.float32)
    q_shared = jnp.dot(s_dot * mix_hi, wdv[48:64, :],
                       preferred_element_type=jnp.float32)
    for c in range(3):
        t_c = jnp.dot(mv[c] * mix_lo, wdv[:16, :],
                      preferred_element_type=jnp.float32)
        r_c = jnp.dot(mv[c] * mix_hi, wdv[48:64, :],
                      preferred_element_type=jnp.float32)
        out_v_c = (t_c + (_SQ3 * u[c]) * p_shared
                   + _SQ75 * (u[c] * q_shared - r_c * (1.0 / 3.0)))
        out_ref[:, 48 + 16 * c:64 + 16 * c] = out_v_c * (1.0 / math.sqrt(64.0))
    out_ref[:, 96:128] = jnp.zeros((g.shape[0], 32), jnp.float32)


def _edge_compute(g, w0, w1, w2, w3, wds, wdv):
    be = BE
    return pl.pallas_call(
        _edge_body,
        grid=(E_PAD // be,),
        in_specs=[
            pl.BlockSpec((be, 128), lambda i: (i, 0)),
            pl.BlockSpec((8, 64), lambda i: (0, 0)),
            pl.BlockSpec((64, 64), lambda i: (0, 0)),
            pl.BlockSpec((64, 64), lambda i: (0, 0)),
            pl.BlockSpec((64, 112), lambda i: (0, 0)),
            pl.BlockSpec((48, 48), lambda i: (0, 0)),
            pl.BlockSpec((64, 16), lambda i: (0, 0)),
        ],
        out_specs=pl.BlockSpec((be, 128), lambda i: (i, 0)),
        out_shape=jax.ShapeDtypeStruct((E_PAD, 128), jnp.float32),
    )(g, w0, w1, w2, w3, wds, wdv)


def _scatter_pass(msg_hbm, recv2d_hbm, zeros_hbm, agg_hbm, col0,
                  slab, idx_v, msg_v):
    tid = lax.axis_index("s")
    row0 = tid * ROWS_PER_TILE
    pltpu.sync_copy(zeros_hbm, slab.at[pl.ds(row0, ROWS_PER_TILE)])
    plsc.subcore_barrier()

    n_g = T_E // KG_S // 16
    base_g = tid * n_g

    def grp(g, carry):
        t0 = (base_g + g) * KG_S
        off = t0 * 128
        pltpu.sync_copy(recv2d_hbm.at[pl.ds(t0, KG_S)], idx_v)
        pltpu.sync_copy(msg_hbm.at[pl.ds(off, KG_S * 128), pl.ds(col0, 16)],
                        msg_v)
        for j in range(KG_S):
            pltpu.sync_copy(msg_v.at[pl.ds(j * 128, 128)],
                            slab.at[idx_v.at[j]], add=True)
        return carry

    lax.fori_loop(0, n_g, grp, 0)
    plsc.subcore_barrier()
    pltpu.sync_copy(slab.at[pl.ds(row0, ROWS_PER_TILE)],
                    agg_hbm.at[pl.ds(row0, ROWS_PER_TILE), pl.ds(col0, 16)])


def _scatter_body(msg, recv2d, zeros, agg, slab, idx_v, msg_v):
    cid = lax.axis_index("c")
    for p in range(3):

        @pl.when(cid == 0)
        def _():
            _scatter_pass(msg, recv2d, zeros, agg, 16 * p,
                          slab, idx_v, msg_v)

        @pl.when(cid == 1)
        def _():
            _scatter_pass(msg, recv2d, zeros, agg, 48 + 16 * p,
                          slab, idx_v, msg_v)


def _sc_scatter(msg, recv2d, zeros_tile):
    mesh = plsc.VectorSubcoreMesh(core_axis_name="c", subcore_axis_name="s")
    fn = pl.kernel(
        _scatter_body,
        out_type=jax.ShapeDtypeStruct((NPAD, 128), jnp.float32),
        mesh=mesh,
        compiler_params=pltpu.CompilerParams(use_tc_tiling_on_sc=False),
        scratch_types=[
            pltpu.VMEM_SHARED((NPAD, 16), jnp.float32),
            pltpu.VMEM((KG_S, 128), jnp.int32),
            pltpu.VMEM((KG_S * 128, 16), jnp.float32),
        ],
    )
    return fn(msg, recv2d, zeros_tile)


def _node_body(agg_ref, nf_ref, wsc_ref, wvc_ref, out_ref):
    nf = nf_ref[...]
    xs = nf[:, :32]
    sp = nf[:, 80:81]
    bsz = nf.shape[0]
    iota = lax.broadcasted_iota(jnp.int32, (bsz, 10), 1).astype(jnp.float32)
    ph = (sp == iota).astype(jnp.float32)

    ys = jnp.dot(xs, wsc_ref[...], preferred_element_type=jnp.float32) \
        * (1.0 / math.sqrt(32.0))
    sks = ph[:, 0:1] * ys[:, 0:48]
    for s in range(1, 10):
        sks = sks + ph[:, s:s + 1] * ys[:, 48 * s:48 * s + 48]

    wvc = wvc_ref[...]
    skv = []
    for c in range(3):
        yv = jnp.dot(nf[:, 32 + 16 * c:48 + 16 * c], wvc,
                     preferred_element_type=jnp.float32) * (1.0 / math.sqrt(16.0))
        acc = ph[:, 0:1] * yv[:, 0:16]
        for s in range(1, 10):
            acc = acc + ph[:, s:s + 1] * yv[:, 16 * s:16 * s + 16]
        skv.append(acc)

    a = agg_ref[...]
    inv_sq = 1.0 / math.sqrt(16.0)
    hs = a[:, 0:48] * inv_sq + sks
    out_ref[:, :32] = jnp.maximum(_silu(hs[:, :32]), 0.0)
    gates = _silu(hs[:, 32:48])

    inter = None
    for c in range(3):
        hv = a[:, 48 + 16 * c:64 + 16 * c] * inv_sq + skv[c]
        ov_c = jnp.maximum(hv * gates, 0.0)
        sel = (lax.broadcasted_iota(jnp.int32, (16, 48), 1)
               == 3 * lax.broadcasted_iota(jnp.int32, (16, 48), 0) + c
               ).astype(jnp.float32)
        term = jnp.dot(ov_c, sel, preferred_element_type=jnp.float32)
        inter = term if inter is None else inter + term
    out_ref[:, 32:80] = inter


def _node_final(agg, nf_prep, ws_cat, wv_cat):
    bn = BN
    return pl.pallas_call(
        _node_body,
        grid=(N // bn,),
        in_specs=[
            pl.BlockSpec((bn, 128), lambda i: (i, 0)),
            pl.BlockSpec((bn, 81), lambda i: (i, 0)),
            pl.BlockSpec((32, 480), lambda i: (0, 0)),
            pl.BlockSpec((16, 160), lambda i: (0, 0)),
        ],
        out_specs=pl.BlockSpec((bn, 80), lambda i: (i, 0)),
        out_shape=jax.ShapeDtypeStruct((N, 80), jnp.float32),
    )(agg, nf_prep, ws_cat, wv_cat)


def kernel(vectors, node_feats, node_specie, radial_embedding, senders,
           receivers, W_up_s, W_up_v, W_mlp0, W_mlp1, W_mlp2, W_mlp3,
           Ws_skip, Wv_skip, Wd_s, Wd_v):
    nf_prep = jnp.concatenate(
        [node_feats[:, :32],
         node_feats[:, 32:].reshape(N, 16, 3).transpose(0, 2, 1).reshape(N, 48),
         node_specie.astype(jnp.float32).reshape(N, 1)],
        axis=1)
    pad_e = E_PAD - E
    senders2d = jnp.concatenate(
        [senders.astype(jnp.int32),
         jnp.zeros((pad_e,), jnp.int32)]).reshape(T_E, 128)
    recv2d = jnp.concatenate(
        [receivers.astype(jnp.int32),
         jnp.full((pad_e,), N, jnp.int32)]).reshape(T_E, 128)
    vr = jnp.concatenate(
        [jnp.pad(vectors, ((0, pad_e), (0, 0))),
         jnp.pad(radial_embedding, ((0, pad_e), (0, 0))),
         jnp.zeros((E_PAD, 5), jnp.float32)], axis=1).reshape(E_PAD // 8, 128)
    ws_cat = Ws_skip.transpose(1, 0, 2).reshape(32, 480)
    wv_cat = Wv_skip.transpose(1, 0, 2).reshape(16, 160)
    zeros_tile = jnp.zeros((ROWS_PER_TILE, 16), jnp.float32)

    u = _linear_up(nf_prep, W_up_s, W_up_v)
    g = _sc_gather(u, senders2d, vr)
    msg = _edge_compute(g, W_mlp0, W_mlp1, W_mlp2, W_mlp3, Wd_s, Wd_v)
    agg = _sc_scatter(msg, recv2d, zeros_tile)
    return _node_final(agg, nf_prep, ws_cat, wv_cat)


# speedup vs baseline: 20.6613x; 1.3183x over previous
import functools
import math

import jax
import jax.numpy as jnp
from jax import lax
from jax.experimental import pallas as pl
from jax.experimental.pallas import tpu as pltpu
from jax.experimental.pallas import tpu_sc as plsc

N = 50000
E = 800000
E_PAD = 819200
T_E = E_PAD // 128
KG_G = 4
KG_S = 8
NPAD = 50048
ROWS_PER_TILE = NPAD // 16

_SQ3 = math.sqrt(3.0)
_SQ75 = math.sqrt(7.5)

BN = 2000
BE = 2048


def _silu(x):
    return x / (1.0 + jnp.exp(-x))


def _up_body(nf_ref, wus_ref, wuv_ref, out_ref):
    nf = nf_ref[...]
    xs = nf[:, :32]
    out_ref[:, :32] = jnp.dot(xs, wus_ref[...],
                              preferred_element_type=jnp.float32) * (1.0 / math.sqrt(32.0))
    wv = wuv_ref[...]
    for c in range(3):
        xv = nf[:, 32 + 16 * c:48 + 16 * c]
        out_ref[:, 32 + 16 * c:48 + 16 * c] = jnp.dot(
            xv, wv, preferred_element_type=jnp.float32) * (1.0 / math.sqrt(16.0))
    out_ref[:, 80:128] = jnp.zeros((nf.shape[0], 48), jnp.float32)


def _linear_up(nf_prep, w_up_s, w_up_v):
    bn = BN
    return pl.pallas_call(
        _up_body,
        grid=(N // bn,),
        in_specs=[
            pl.BlockSpec((bn, 81), lambda i: (i, 0)),
            pl.BlockSpec((32, 32), lambda i: (0, 0)),
            pl.BlockSpec((16, 16), lambda i: (0, 0)),
        ],
        out_specs=pl.BlockSpec((bn, 128), lambda i: (i, 0)),
        out_shape=jax.ShapeDtypeStruct((N, 128), jnp.float32),
    )(nf_prep, w_up_s, w_up_v)


def _gather_body(u_hbm, idx2d_hbm, vr_hbm, out_hbm, idx_v, rows_v, vr_v, sem):
    cid = lax.axis_index("c")
    sid = lax.axis_index("s")
    wid = sid * 2 + cid
    n_g = T_E // KG_G // 32
    base_g = wid * n_g

    def grp(g, carry):
        t0 = (base_g + g) * KG_G
        off = t0 * 128
        pltpu.sync_copy(idx2d_hbm.at[pl.ds(t0, KG_G)], idx_v)
        cps = [
            pltpu.async_copy(u_hbm.at[idx_v.at[j]],
                             rows_v.at[pl.ds(j * 128, 128)], sem)
            for j in range(KG_G)
        ]
        pltpu.sync_copy(vr_hbm.at[pl.ds(off // 8, KG_G * 16)], vr_v)
        for cp in cps:
            cp.wait()

        def inj(r, carry2):
            for c in range(8):
                rows_v[r * 8 + c, pl.ds(80, 16)] = vr_v[r, pl.ds(16 * c, 16)]
            return carry2

        lax.fori_loop(0, KG_G * 16, inj, 0)
        pltpu.sync_copy(rows_v, out_hbm.at[pl.ds(off, KG_G * 128)])
        return carry

    lax.fori_loop(0, n_g, grp, 0)


def _sc_gather(u, senders2d, vr):
    mesh = plsc.VectorSubcoreMesh(core_axis_name="c", subcore_axis_name="s")
    fn = pl.kernel(
        _gather_body,
        out_type=jax.ShapeDtypeStruct((E_PAD, 128), jnp.float32),
        mesh=mesh,
        compiler_params=pltpu.CompilerParams(use_tc_tiling_on_sc=False),
        scratch_types=[
            pltpu.VMEM((KG_G, 128), jnp.int32),
            pltpu.VMEM((KG_G * 128, 128), jnp.float32),
            pltpu.VMEM((KG_G * 16, 128), jnp.float32),
            pltpu.SemaphoreType.DMA,
        ],
    )
    return fn(u, senders2d, vr)


def _edge_body(g_ref, w0_ref, w1_ref, w2_ref, w3_ref, wds_ref, wdv_ref,
               s3_ref, til_ref, wlo3_ref, whi3_ref, one112_ref, out_ref):
    g = g_ref[...]
    ms = g[:, :32]
    mv_all = g[:, 32:80]
    v3 = g[:, 80:83]
    vsq = v3 * v3
    r2 = vsq[:, 0:1] + vsq[:, 1:2] + vsq[:, 2:3]
    nz = r2 > 0.0
    inv = jnp.where(nz, lax.rsqrt(jnp.where(nz, r2, 1.0)), 0.0)
    u3 = v3 * inv

    h = g[:, 83:91]
    h = _silu(jnp.dot(h, w0_ref[...], preferred_element_type=jnp.float32)
              * (1.0 / math.sqrt(8.0)))
    h = _silu(jnp.dot(h, w1_ref[...], preferred_element_type=jnp.float32) * 0.125)
    h = _silu(jnp.dot(h, w2_ref[...], preferred_element_type=jnp.float32) * 0.125)
    mix = jnp.dot(h, w3_ref[...], preferred_element_type=jnp.float32) * 0.125
    nzf = jnp.where(nz, 1.0, 0.0)
    mix = mix * jnp.dot(nzf, one112_ref[...], preferred_element_type=jnp.float32)

    mix_s1 = mix[:, :32]
    mix_s2 = mix[:, 32:48]
    wds = wds_ref[...]
    wdv = wdv_ref[...]
    til = til_ref[...]

    u48 = jnp.dot(u3, s3_ref[...], preferred_element_type=jnp.float32)
    sm = mv_all * u48
    s_dot = sm[:, :16] + sm[:, 16:32] + sm[:, 32:48]

    outs = (jnp.dot(ms * mix_s1, wds[:32, :], preferred_element_type=jnp.float32)
            + jnp.dot((_SQ3 * s_dot) * mix_s2, wds[32:48, :],
                      preferred_element_type=jnp.float32)) * (1.0 / math.sqrt(48.0))
    out_ref[:, 0:48] = outs

    mixlo3 = jnp.dot(mix[:, 48:64], til, preferred_element_type=jnp.float32)
    mixhi3 = jnp.dot(mix[:, 96:112], til, preferred_element_type=jnp.float32)
    t_all = jnp.dot(mv_all * mixlo3, wlo3_ref[...],
                    preferred_element_type=jnp.float32)
    r_all = jnp.dot(mv_all * mixhi3, whi3_ref[...],
                    preferred_element_type=jnp.float32)
    p = jnp.dot(ms * mix[:, 64:96], wdv[16:48, :],
                preferred_element_type=jnp.float32)
    q = jnp.dot(s_dot * mix[:, 96:112], wdv[48:64, :],
                preferred_element_type=jnp.float32)
    pq3 = jnp.dot(_SQ3 * p + _SQ75 * q, til,
                  preferred_element_type=jnp.float32)
    ov = (t_all + u48 * pq3 - (_SQ75 / 3.0) * r_all) * (1.0 / math.sqrt(64.0))
    out_ref[:, 48:96] = ov
    out_ref[:, 96:128] = jnp.zeros((g.shape[0], 32), jnp.float32)


def _edge_compute(g, w0, w1, w2, w3, wds, wdv, s3, til, wlo3, whi3, one112):
    be = BE
    return pl.pallas_call(
        _edge_body,
        grid=(E_PAD // be,),
        in_specs=[
            pl.BlockSpec((be, 128), lambda i: (i, 0)),
            pl.BlockSpec((8, 64), lambda i: (0, 0)),
            pl.BlockSpec((64, 64), lambda i: (0, 0)),
            pl.BlockSpec((64, 64), lambda i: (0, 0)),
            pl.BlockSpec((64, 112), lambda i: (0, 0)),
            pl.BlockSpec((48, 48), lambda i: (0, 0)),
            pl.BlockSpec((64, 16), lambda i: (0, 0)),
            pl.BlockSpec((3, 48), lambda i: (0, 0)),
            pl.BlockSpec((16, 48), lambda i: (0, 0)),
            pl.BlockSpec((48, 48), lambda i: (0, 0)),
            pl.BlockSpec((48, 48), lambda i: (0, 0)),
            pl.BlockSpec((1, 112), lambda i: (0, 0)),
        ],
        out_specs=pl.BlockSpec((be, 128), lambda i: (i, 0)),
        out_shape=jax.ShapeDtypeStruct((E_PAD, 128), jnp.float32),
    )(g, w0, w1, w2, w3, wds, wdv, s3, til, wlo3, whi3, one112)


def _scatter_pass(msg_hbm, recv2d_hbm, zeros_hbm, agg_hbm, col0,
                  slab, idx_v, msg_v):
    tid = lax.axis_index("s")
    row0 = tid * ROWS_PER_TILE
    pltpu.sync_copy(zeros_hbm, slab.at[pl.ds(row0, ROWS_PER_TILE)])
    plsc.subcore_barrier()

    n_g = T_E // KG_S // 16
    base_g = tid * n_g

    def grp(g, carry):
        t0 = (base_g + g) * KG_S
        off = t0 * 128
        pltpu.sync_copy(recv2d_hbm.at[pl.ds(t0, KG_S)], idx_v)
        pltpu.sync_copy(msg_hbm.at[pl.ds(off, KG_S * 128), pl.ds(col0, 16)],
                        msg_v)
        for j in range(KG_S):
            pltpu.sync_copy(msg_v.at[pl.ds(j * 128, 128)],
                            slab.at[idx_v.at[j]], add=True)
        return carry

    lax.fori_loop(0, n_g, grp, 0)
    plsc.subcore_barrier()
    pltpu.sync_copy(slab.at[pl.ds(row0, ROWS_PER_TILE)],
                    agg_hbm.at[pl.ds(row0, ROWS_PER_TILE), pl.ds(col0, 16)])


def _scatter_body(msg, recv2d, zeros, agg, slab, idx_v, msg_v):
    cid = lax.axis_index("c")
    for p in range(3):

        @pl.when(cid == 0)
        def _():
            _scatter_pass(msg, recv2d, zeros, agg, 16 * p,
                          slab, idx_v, msg_v)

        @pl.when(cid == 1)
        def _():
            _scatter_pass(msg, recv2d, zeros, agg, 48 + 16 * p,
                          slab, idx_v, msg_v)


def _sc_scatter(msg, recv2d, zeros_tile):
    mesh = plsc.VectorSubcoreMesh(core_axis_name="c", subcore_axis_name="s")
    fn = pl.kernel(
        _scatter_body,
        out_type=jax.ShapeDtypeStruct((NPAD, 128), jnp.float32),
        mesh=mesh,
        compiler_params=pltpu.CompilerParams(use_tc_tiling_on_sc=False),
        scratch_types=[
            pltpu.VMEM_SHARED((NPAD, 16), jnp.float32),
            pltpu.VMEM((KG_S, 128), jnp.int32),
            pltpu.VMEM((KG_S * 128, 16), jnp.float32),
        ],
    )
    return fn(msg, recv2d, zeros_tile)


def _node_body(agg_ref, nf_ref, rep32_ref, b320_ref, rep16_ref, b160_ref,
               wst_ref, wvst_ref, out_ref):
    nf = nf_ref[...]
    xs = nf[:, :32]
    sp = nf[:, 80:81]
    bsz = nf.shape[0]
    iota = lax.broadcasted_iota(jnp.int32, (bsz, 10), 1).astype(jnp.float32)
    ph = (sp == iota).astype(jnp.float32)

    pa = jnp.dot(ph, b320_ref[...], preferred_element_type=jnp.float32)
    xa = jnp.dot(xs, rep32_ref[...], preferred_element_type=jnp.float32)
    sks = jnp.dot(xa * pa, wst_ref[...],
                  preferred_element_type=jnp.float32) * (1.0 / math.sqrt(32.0))

    pv = jnp.dot(ph, b160_ref[...], preferred_element_type=jnp.float32)
    wvst = wvst_ref[...]
    rep16 = rep16_ref[...]
    skv = []
    for c in range(3):
        xv = jnp.dot(nf[:, 32 + 16 * c:48 + 16 * c], rep16,
                     preferred_element_type=jnp.float32)
        skv.append(jnp.dot(xv * pv, wvst,
                           preferred_element_type=jnp.float32) * (1.0 / math.sqrt(16.0)))

    a = agg_ref[...]
    inv_sq = 1.0 / math.sqrt(16.0)
    hs = a[:, 0:48] * inv_sq + sks
    out_ref[:, :32] = jnp.maximum(_silu(hs[:, :32]), 0.0)
    gates = _silu(hs[:, 32:48])

    inter = None
    for c in range(3):
        hv = a[:, 48 + 16 * c:64 + 16 * c] * inv_sq + skv[c]
        ov_c = jnp.maximum(hv * gates, 0.0)
        sel = (lax.broadcasted_iota(jnp.int32, (16, 48), 1)
               == 3 * lax.broadcasted_iota(jnp.int32, (16, 48), 0) + c
               ).astype(jnp.float32)
        term = jnp.dot(ov_c, sel, preferred_element_type=jnp.float32)
        inter = term if inter is None else inter + term
    out_ref[:, 32:80] = inter


def _node_final(agg, nf_prep, rep32, b320, rep16, b160, wstack, wvstack):
    bn = BN
    return pl.pallas_call(
        _node_body,
        grid=(N // bn,),
        in_specs=[
            pl.BlockSpec((bn, 128), lambda i: (i, 0)),
            pl.BlockSpec((bn, 81), lambda i: (i, 0)),
            pl.BlockSpec((32, 320), lambda i: (0, 0)),
            pl.BlockSpec((10, 320), lambda i: (0, 0)),
            pl.BlockSpec((16, 160), lambda i: (0, 0)),
            pl.BlockSpec((10, 160), lambda i: (0, 0)),
            pl.BlockSpec((320, 48), lambda i: (0, 0)),
            pl.BlockSpec((160, 16), lambda i: (0, 0)),
        ],
        out_specs=pl.BlockSpec((bn, 80), lambda i: (i, 0)),
        out_shape=jax.ShapeDtypeStruct((N, 80), jnp.float32),
    )(agg, nf_prep, rep32, b320, rep16, b160, wstack, wvstack)


def kernel(vectors, node_feats, node_specie, radial_embedding, senders,
           receivers, W_up_s, W_up_v, W_mlp0, W_mlp1, W_mlp2, W_mlp3,
           Ws_skip, Wv_skip, Wd_s, Wd_v):
    nf_prep = jnp.concatenate(
        [node_feats[:, :32],
         node_feats[:, 32:].reshape(N, 16, 3).transpose(0, 2, 1).reshape(N, 48),
         node_specie.astype(jnp.float32).reshape(N, 1)],
        axis=1)
    pad_e = E_PAD - E
    senders2d = jnp.concatenate(
        [senders.astype(jnp.int32),
         jnp.zeros((pad_e,), jnp.int32)]).reshape(T_E, 128)
    recv2d = jnp.concatenate(
        [receivers.astype(jnp.int32),
         jnp.full((pad_e,), N, jnp.int32)]).reshape(T_E, 128)
    vr = jnp.concatenate(
        [jnp.pad(vectors, ((0, pad_e), (0, 0))),
         jnp.pad(radial_embedding, ((0, pad_e), (0, 0))),
         jnp.zeros((E_PAD, 5), jnp.float32)], axis=1).reshape(E_PAD // 8, 128)
    eye16 = jnp.eye(16, dtype=jnp.float32)
    til = jnp.tile(eye16, (1, 3))
    s3 = jnp.kron(jnp.eye(3, dtype=jnp.float32),
                  jnp.ones((1, 16), jnp.float32))
    wlo3 = jnp.kron(jnp.eye(3, dtype=jnp.float32), Wd_v[:16, :])
    whi3 = jnp.kron(jnp.eye(3, dtype=jnp.float32), Wd_v[48:64, :])
    one112 = jnp.ones((1, 112), jnp.float32)
    rep32 = jnp.tile(jnp.eye(32, dtype=jnp.float32), (1, 10))
    b320 = jnp.kron(jnp.eye(10, dtype=jnp.float32),
                    jnp.ones((1, 32), jnp.float32))
    rep16 = jnp.tile(eye16, (1, 10))
    b160 = jnp.kron(jnp.eye(10, dtype=jnp.float32),
                    jnp.ones((1, 16), jnp.float32))
    wstack = Ws_skip.reshape(320, 48)
    wvstack = Wv_skip.reshape(160, 16)
    zeros_tile = jnp.zeros((ROWS_PER_TILE, 16), jnp.float32)

    u = _linear_up(nf_prep, W_up_s, W_up_v)
    g = _sc_gather(u, senders2d, vr)
    msg = _edge_compute(g, W_mlp0, W_mlp1, W_mlp2, W_mlp3, Wd_s, Wd_v,
                        s3, til, wlo3, whi3, one112)
    agg = _sc_scatter(msg, recv2d, zeros_tile)
    return _node_final(agg, nf_prep, rep32, b320, rep16, b160, wstack, wvstack)


# speedup vs baseline: 23.8022x; 1.1520x over previous
import functools
import math

import jax
import jax.numpy as jnp
from jax import lax
from jax.experimental import pallas as pl
from jax.experimental.pallas import tpu as pltpu
from jax.experimental.pallas import tpu_sc as plsc

N = 50000
E = 800000
E_PAD = 819200
T_E = E_PAD // 128
KG_G = 2
KG_S = 8
G0_TILE = 146
G1_TILE = 54
NPAD = 50048
ROWS_PER_TILE = NPAD // 16

_SQ3 = math.sqrt(3.0)
_SQ75 = math.sqrt(7.5)

BN = 2000
BE = 2048


def _silu(x):
    return x / (1.0 + jnp.exp(-x))


def _up_body(nf_ref, wus_ref, wuv_ref, out_ref):
    nf = nf_ref[...]
    xs = nf[:, :32]
    out_ref[:, :32] = jnp.dot(xs, wus_ref[...],
                              preferred_element_type=jnp.float32) * (1.0 / math.sqrt(32.0))
    wv = wuv_ref[...]
    for c in range(3):
        xv = nf[:, 32 + 16 * c:48 + 16 * c]
        out_ref[:, 32 + 16 * c:48 + 16 * c] = jnp.dot(
            xv, wv, preferred_element_type=jnp.float32) * (1.0 / math.sqrt(16.0))


def _linear_up(nf_prep, w_up_s, w_up_v):
    bn = BN
    return pl.pallas_call(
        _up_body,
        grid=(N // bn,),
        in_specs=[
            pl.BlockSpec((bn, 81), lambda i: (i, 0)),
            pl.BlockSpec((32, 32), lambda i: (0, 0)),
            pl.BlockSpec((16, 16), lambda i: (0, 0)),
        ],
        out_specs=pl.BlockSpec((bn, 80), lambda i: (i, 0)),
        out_shape=jax.ShapeDtypeStruct((N, 80), jnp.float32),
    )(nf_prep, w_up_s, w_up_v)


def _gather_body(u_hbm, idx2d_hbm, vr_hbm, out_hbm,
                 idx0, idx1, rows0, rows1, vrw0, vrw1, vrr0, vrr1,
                 lsem0, lsem1, gsem0, gsem1, ssem0, ssem1):
    cid = lax.axis_index("c")
    sid = lax.axis_index("s")
    n_g = jnp.where(cid == 0, G0_TILE, G1_TILE)
    base_g = jnp.where(cid == 0, sid * G0_TILE, 16 * G0_TILE + sid * G1_TILE)
    idx_v = [idx0, idx1]
    rows_v = [rows0, rows1]
    vrw_v = [vrw0, vrw1]
    vrr_v = [vrr0, vrr1]
    lsem = [lsem0, lsem1]
    gsem = [gsem0, gsem1]
    ssem = [ssem0, ssem1]

    def fire_loads(g, b):
        t0 = (base_g + g) * KG_G
        pltpu.async_copy(idx2d_hbm.at[pl.ds(t0, KG_G)], idx_v[b], lsem[b])
        pltpu.async_copy(vr_hbm.at[pl.ds(t0 * 16, KG_G * 16)], vrw_v[b], lsem[b])

    def drain_loads(b):
        pltpu.make_async_copy(idx2d_hbm.at[pl.ds(0, KG_G)], idx_v[b], lsem[b]).wait()
        pltpu.make_async_copy(vr_hbm.at[pl.ds(0, KG_G * 16)], vrw_v[b], lsem[b]).wait()

    def fire_gathers(b):
        for j in range(KG_G):
            pltpu.async_copy(u_hbm.at[idx_v[b].at[j]],
                             rows_v[b].at[pl.ds(j * 128, 128)], gsem[b])

    def drain_gathers(b):
        for j in range(KG_G):
            pltpu.make_async_copy(u_hbm.at[idx_v[b].at[j]],
                                  rows_v[b].at[pl.ds(j * 128, 128)], gsem[b]).wait()

    def fire_store(g, b):
        off = (base_g + g) * KG_G * 128
        pltpu.async_copy(rows_v[b],
                         out_hbm.at[pl.ds(off, KG_G * 128), pl.ds(0, 80)], ssem[b])
        pltpu.async_copy(vrr_v[b],
                         out_hbm.at[pl.ds(off, KG_G * 128), pl.ds(80, 16)], ssem[b])

    def drain_store(b):
        pltpu.make_async_copy(rows_v[b],
                              out_hbm.at[pl.ds(0, KG_G * 128), pl.ds(0, 80)],
                              ssem[b]).wait()
        pltpu.make_async_copy(vrr_v[b],
                              out_hbm.at[pl.ds(0, KG_G * 128), pl.ds(80, 16)],
                              ssem[b]).wait()

    def inject(b):
        def inj(r, carry2):
            for c in range(8):
                vrr_v[b][r * 8 + c, :] = vrw_v[b][r, pl.ds(16 * c, 16)]
            return carry2
        lax.fori_loop(0, KG_G * 16, inj, 0)

    fire_loads(0, 0)
    fire_loads(1, 1)

    def pair(p, carry):
        for b in range(2):
            g = 2 * p + b

            @pl.when(p >= 1)
            def _():
                drain_store(b)

            drain_loads(b)
            fire_gathers(b)
        for b in range(2):
            g = 2 * p + b
            drain_gathers(b)
            inject(b)
            fire_store(g, b)

            @pl.when(g + 2 < n_g)
            def _():
                fire_loads(g + 2, b)
        return carry

    lax.fori_loop(0, n_g // 2, pair, 0)
    drain_store(0)
    drain_store(1)


def _sc_gather(u, senders2d, vr):
    mesh = plsc.VectorSubcoreMesh(core_axis_name="c", subcore_axis_name="s")
    fn = pl.kernel(
        _gather_body,
        out_type=jax.ShapeDtypeStruct((E_PAD, 128), jnp.float32),
        mesh=mesh,
        compiler_params=pltpu.CompilerParams(use_tc_tiling_on_sc=False),
        scratch_types=[
            pltpu.VMEM((KG_G, 128), jnp.int32),
            pltpu.VMEM((KG_G, 128), jnp.int32),
            pltpu.VMEM((KG_G * 128, 80), jnp.float32),
            pltpu.VMEM((KG_G * 128, 80), jnp.float32),
            pltpu.VMEM((KG_G * 16, 128), jnp.float32),
            pltpu.VMEM((KG_G * 16, 128), jnp.float32),
            pltpu.VMEM((KG_G * 128, 16), jnp.float32),
            pltpu.VMEM((KG_G * 128, 16), jnp.float32),
        ] + [pltpu.SemaphoreType.DMA] * 6,
    )
    return fn(u, senders2d, vr)


def _edge_body(g_ref, w0_ref, w1_ref, w2_ref, w3_ref, wds_ref, wdv_ref,
               s3_ref, til_ref, wlo3_ref, whi3_ref, one112_ref, out_ref):
    g = g_ref[...]
    ms = g[:, :32]
    mv_all = g[:, 32:80]
    v3 = g[:, 80:83]
    vsq = v3 * v3
    r2 = vsq[:, 0:1] + vsq[:, 1:2] + vsq[:, 2:3]
    nz = r2 > 0.0
    inv = jnp.where(nz, lax.rsqrt(jnp.where(nz, r2, 1.0)), 0.0)
    u3 = v3 * inv

    h = g[:, 83:91]
    h = _silu(jnp.dot(h, w0_ref[...], preferred_element_type=jnp.float32)
              * (1.0 / math.sqrt(8.0)))
    h = _silu(jnp.dot(h, w1_ref[...], preferred_element_type=jnp.float32) * 0.125)
    h = _silu(jnp.dot(h, w2_ref[...], preferred_element_type=jnp.float32) * 0.125)
    mix = jnp.dot(h, w3_ref[...], preferred_element_type=jnp.float32) * 0.125
    nzf = jnp.where(nz, 1.0, 0.0)
    mix = mix * jnp.dot(nzf, one112_ref[...], preferred_element_type=jnp.float32)

    mix_s1 = mix[:, :32]
    mix_s2 = mix[:, 32:48]
    wds = wds_ref[...]
    wdv = wdv_ref[...]
    til = til_ref[...]

    u48 = jnp.dot(u3, s3_ref[...], preferred_element_type=jnp.float32)
    sm = mv_all * u48
    s_dot = sm[:, :16] + sm[:, 16:32] + sm[:, 32:48]

    outs = (jnp.dot(ms * mix_s1, wds[:32, :], preferred_element_type=jnp.float32)
            + jnp.dot((_SQ3 * s_dot) * mix_s2, wds[32:48, :],
                      preferred_element_type=jnp.float32)) * (1.0 / math.sqrt(48.0))
    out_ref[:, 0:48] = outs

    mixlo3 = jnp.dot(mix[:, 48:64], til, preferred_element_type=jnp.float32)
    mixhi3 = jnp.dot(mix[:, 96:112], til, preferred_element_type=jnp.float32)
    t_all = jnp.dot(mv_all * mixlo3, wlo3_ref[...],
                    preferred_element_type=jnp.float32)
    r_all = jnp.dot(mv_all * mixhi3, whi3_ref[...],
                    preferred_element_type=jnp.float32)
    p = jnp.dot(ms * mix[:, 64:96], wdv[16:48, :],
                preferred_element_type=jnp.float32)
    q = jnp.dot(s_dot * mix[:, 96:112], wdv[48:64, :],
                preferred_element_type=jnp.float32)
    pq3 = jnp.dot(_SQ3 * p + _SQ75 * q, til,
                  preferred_element_type=jnp.float32)
    ov = (t_all + u48 * pq3 - (_SQ75 / 3.0) * r_all) * (1.0 / math.sqrt(64.0))
    out_ref[:, 48:96] = ov
    out_ref[:, 96:128] = jnp.zeros((g.shape[0], 32), jnp.float32)


def _edge_compute(g, w0, w1, w2, w3, wds, wdv, s3, til, wlo3, whi3, one112):
    be = BE
    return pl.pallas_call(
        _edge_body,
        grid=(E_PAD // be,),
        in_specs=[
            pl.BlockSpec((be, 128), lambda i: (i, 0)),
            pl.BlockSpec((8, 64), lambda i: (0, 0)),
            pl.BlockSpec((64, 64), lambda i: (0, 0)),
            pl.BlockSpec((64, 64), lambda i: (0, 0)),
            pl.BlockSpec((64, 112), lambda i: (0, 0)),
            pl.BlockSpec((48, 48), lambda i: (0, 0)),
            pl.BlockSpec((64, 16), lambda i: (0, 0)),
            pl.BlockSpec((3, 48), lambda i: (0, 0)),
            pl.BlockSpec((16, 48), lambda i: (0, 0)),
            pl.BlockSpec((48, 48), lambda i: (0, 0)),
            pl.BlockSpec((48, 48), lambda i: (0, 0)),
            pl.BlockSpec((1, 112), lambda i: (0, 0)),
        ],
        out_specs=pl.BlockSpec((be, 128), lambda i: (i, 0)),
        out_shape=jax.ShapeDtypeStruct((E_PAD, 128), jnp.float32),
    )(g, w0, w1, w2, w3, wds, wdv, s3, til, wlo3, whi3, one112)


def _scatter_pass(msg_hbm, recv2d_hbm, zeros_hbm, agg_hbm, col0,
                  idx_v, msg_v, lsem, asem, slab):
    tid = lax.axis_index("s")
    row0 = tid * ROWS_PER_TILE
    pltpu.sync_copy(zeros_hbm, slab.at[pl.ds(row0, ROWS_PER_TILE)])
    plsc.subcore_barrier()

    n_g = T_E // KG_S // 16
    base_g = tid * n_g

    def fire_loads(g, b):
        t0 = (base_g + g) * KG_S
        pltpu.async_copy(recv2d_hbm.at[pl.ds(t0, KG_S)], idx_v[b], lsem[b])
        pltpu.async_copy(msg_hbm.at[pl.ds(t0 * 128, KG_S * 128),
                                    pl.ds(col0, 16)], msg_v[b], lsem[b])

    def drain_loads(b):
        pltpu.make_async_copy(recv2d_hbm.at[pl.ds(0, KG_S)], idx_v[b],
                              lsem[b]).wait()
        pltpu.make_async_copy(msg_hbm.at[pl.ds(0, KG_S * 128), pl.ds(col0, 16)],
                              msg_v[b], lsem[b]).wait()

    def fire_adds(b):
        for j in range(KG_S):
            pltpu.async_copy(msg_v[b].at[pl.ds(j * 128, 128)],
                             slab.at[idx_v[b].at[j]], asem[b], add=True)

    def drain_adds(b):
        for j in range(KG_S):
            pltpu.make_async_copy(msg_v[b].at[pl.ds(j * 128, 128)],
                                  slab.at[idx_v[b].at[j]], asem[b]).wait()

    fire_loads(0, 0)
    fire_loads(1, 1)

    def pair(p, carry):
        for b in range(2):
            drain_loads(b)
            fire_adds(b)
        for b in range(2):
            g = 2 * p + b
            drain_adds(b)

            @pl.when(g + 2 < n_g)
            def _():
                fire_loads(g + 2, b)
        return carry

    lax.fori_loop(0, n_g // 2, pair, 0)
    plsc.subcore_barrier()
    pltpu.sync_copy(slab.at[pl.ds(row0, ROWS_PER_TILE)],
                    agg_hbm.at[pl.ds(row0, ROWS_PER_TILE), pl.ds(col0, 16)])


def _scatter_body(msg, recv2d, zeros, agg,
                  slab, idx0, idx1, msg0, msg1, lsem0, lsem1, asem0, asem1):
    cid = lax.axis_index("c")
    idx_v = [idx0, idx1]
    msg_v = [msg0, msg1]
    lsem = [lsem0, lsem1]
    asem = [asem0, asem1]
    for p in range(3):

        @pl.when(cid == 0)
        def _():
            _scatter_pass(msg, recv2d, zeros, agg, 16 * p,
                          idx_v, msg_v, lsem, asem, slab)

        @pl.when(cid == 1)
        def _():
            _scatter_pass(msg, recv2d, zeros, agg, 48 + 16 * p,
                          idx_v, msg_v, lsem, asem, slab)


def _sc_scatter(msg, recv2d, zeros_tile):
    mesh = plsc.VectorSubcoreMesh(core_axis_name="c", subcore_axis_name="s")
    fn = pl.kernel(
        _scatter_body,
        out_type=jax.ShapeDtypeStruct((NPAD, 128), jnp.float32),
        mesh=mesh,
        compiler_params=pltpu.CompilerParams(use_tc_tiling_on_sc=False),
        scratch_types=[
            pltpu.VMEM_SHARED((NPAD, 16), jnp.float32),
            pltpu.VMEM((KG_S, 128), jnp.int32),
            pltpu.VMEM((KG_S, 128), jnp.int32),
            pltpu.VMEM((KG_S * 128, 16), jnp.float32),
            pltpu.VMEM((KG_S * 128, 16), jnp.float32),
        ] + [pltpu.SemaphoreType.DMA] * 4,
    )
    return fn(msg, recv2d, zeros_tile)


def _node_body(agg_ref, nf_ref, rep32_ref, b320_ref, rep16_ref, b160_ref,
               wst_ref, wvst_ref, out_ref):
    nf = nf_ref[...]
    xs = nf[:, :32]
    sp = nf[:, 80:81]
    bsz = nf.shape[0]
    iota = lax.broadcasted_iota(jnp.int32, (bsz, 10), 1).astype(jnp.float32)
    ph = (sp == iota).astype(jnp.float32)

    pa = jnp.dot(ph, b320_ref[...], preferred_element_type=jnp.float32)
    xa = jnp.dot(xs, rep32_ref[...], preferred_element_type=jnp.float32)
    sks = jnp.dot(xa * pa, wst_ref[...],
                  preferred_element_type=jnp.float32) * (1.0 / math.sqrt(32.0))

    pv = jnp.dot(ph, b160_ref[...], preferred_element_type=jnp.float32)
    wvst = wvst_ref[...]
    rep16 = rep16_ref[...]
    skv = []
    for c in range(3):
        xv = jnp.dot(nf[:, 32 + 16 * c:48 + 16 * c], rep16,
                     preferred_element_type=jnp.float32)
        skv.append(jnp.dot(xv * pv, wvst,
                           preferred_element_type=jnp.float32) * (1.0 / math.sqrt(16.0)))

    a = agg_ref[...]
    inv_sq = 1.0 / math.sqrt(16.0)
    hs = a[:, 0:48] * inv_sq + sks
    out_ref[:, :32] = jnp.maximum(_silu(hs[:, :32]), 0.0)
    gates = _silu(hs[:, 32:48])

    inter = None
    for c in range(3):
        hv = a[:, 48 + 16 * c:64 + 16 * c] * inv_sq + skv[c]
        ov_c = jnp.maximum(hv * gates, 0.0)
        sel = (lax.broadcasted_iota(jnp.int32, (16, 48), 1)
               == 3 * lax.broadcasted_iota(jnp.int32, (16, 48), 0) + c
               ).astype(jnp.float32)
        term = jnp.dot(ov_c, sel, preferred_element_type=jnp.float32)
        inter = term if inter is None else inter + term
    out_ref[:, 32:80] = inter


def _node_final(agg, nf_prep, rep32, b320, rep16, b160, wstack, wvstack):
    bn = BN
    return pl.pallas_call(
        _node_body,
        grid=(N // bn,),
        in_specs=[
            pl.BlockSpec((bn, 128), lambda i: (i, 0)),
            pl.BlockSpec((bn, 81), lambda i: (i, 0)),
            pl.BlockSpec((32, 320), lambda i: (0, 0)),
            pl.BlockSpec((10, 320), lambda i: (0, 0)),
            pl.BlockSpec((16, 160), lambda i: (0, 0)),
            pl.BlockSpec((10, 160), lambda i: (0, 0)),
            pl.BlockSpec((320, 48), lambda i: (0, 0)),
            pl.BlockSpec((160, 16), lambda i: (0, 0)),
        ],
        out_specs=pl.BlockSpec((bn, 80), lambda i: (i, 0)),
        out_shape=jax.ShapeDtypeStruct((N, 80), jnp.float32),
    )(agg, nf_prep, rep32, b320, rep16, b160, wstack, wvstack)


def kernel(vectors, node_feats, node_specie, radial_embedding, senders,
           receivers, W_up_s, W_up_v, W_mlp0, W_mlp1, W_mlp2, W_mlp3,
           Ws_skip, Wv_skip, Wd_s, Wd_v):
    nf_prep = jnp.concatenate(
        [node_feats[:, :32],
         node_feats[:, 32:].reshape(N, 16, 3).transpose(0, 2, 1).reshape(N, 48),
         node_specie.astype(jnp.float32).reshape(N, 1)],
        axis=1)
    pad_e = E_PAD - E
    senders2d = jnp.concatenate(
        [senders.astype(jnp.int32),
         jnp.zeros((pad_e,), jnp.int32)]).reshape(T_E, 128)
    recv2d = jnp.concatenate(
        [receivers.astype(jnp.int32),
         jnp.full((pad_e,), N, jnp.int32)]).reshape(T_E, 128)
    vr = jnp.concatenate(
        [jnp.pad(vectors, ((0, pad_e), (0, 0))),
         jnp.pad(radial_embedding, ((0, pad_e), (0, 0))),
         jnp.zeros((E_PAD, 5), jnp.float32)], axis=1).reshape(E_PAD // 8, 128)
    eye16 = jnp.eye(16, dtype=jnp.float32)
    til = jnp.tile(eye16, (1, 3))
    s3 = jnp.kron(jnp.eye(3, dtype=jnp.float32),
                  jnp.ones((1, 16), jnp.float32))
    wlo3 = jnp.kron(jnp.eye(3, dtype=jnp.float32), Wd_v[:16, :])
    whi3 = jnp.kron(jnp.eye(3, dtype=jnp.float32), Wd_v[48:64, :])
    one112 = jnp.ones((1, 112), jnp.float32)
    rep32 = jnp.tile(jnp.eye(32, dtype=jnp.float32), (1, 10))
    b320 = jnp.kron(jnp.eye(10, dtype=jnp.float32),
                    jnp.ones((1, 32), jnp.float32))
    rep16 = jnp.tile(eye16, (1, 10))
    b160 = jnp.kron(jnp.eye(10, dtype=jnp.float32),
                    jnp.ones((1, 16), jnp.float32))
    wstack = Ws_skip.reshape(320, 48)
    wvstack = Wv_skip.reshape(160, 16)
    zeros_tile = jnp.zeros((ROWS_PER_TILE, 16), jnp.float32)

    u = _linear_up(nf_prep, W_up_s, W_up_v)
    g = _sc_gather(u, senders2d, vr)
    msg = _edge_compute(g, W_mlp0, W_mlp1, W_mlp2, W_mlp3, Wd_s, Wd_v,
                        s3, til, wlo3, whi3, one112)
    agg = _sc_scatter(msg, recv2d, zeros_tile)
    return _node_final(agg, nf_prep, rep32, b320, rep16, b160, wstack, wvstack)
